# Initial kernel scaffold; baseline (speedup 1.0000x reference)
import jax
import jax.numpy as jnp
from jax.experimental import pallas as pl


def kernel(input_features, theta_params, phi_params, W1, b1, W2, b2, W3, b3):
    raise NotImplementedError("write your pallas kernel here")


# speedup vs baseline: 8.7643x; 8.7643x over previous
import jax
import jax.numpy as jnp
from jax.experimental import pallas as pl
from jax.experimental.pallas import tpu as pltpu
from functools import partial

N_QUBITS = 15
N_LAYERS = 3
DIM = 2 ** N_QUBITS
B = 128
PAIRS = [(i, (i + 1) % N_QUBITS) for i in range(N_QUBITS)]


def _bf(x):
    return x.astype(jnp.bfloat16).astype(jnp.float32)


CH = 256
CH_LOG = 8
NCH = DIM // CH


def _quantum_body(cos_ref, sin_ref, out_ref):
    fori = jax.lax.fori_loop

    def cs(g):
        return _bf(cos_ref[g]), _bf(sin_ref[g])

    def ry_small(q, g):
        c, sn = cs(g)
        s = 1 << q

        def body(t, _):
            r0 = t * CH
            x = out_ref[pl.ds(r0, CH), :]
            v = x.reshape(CH // (2 * s), 2, s, B)
            a = _bf(v[:, 0])
            b = _bf(v[:, 1])
            na = c * a - sn * b
            nb = sn * a + c * b
            out_ref[pl.ds(r0, CH), :] = jnp.stack(
                [na, nb], axis=1).reshape(CH, B)
            return 0

        fori(0, NCH, body, 0)

    def ry_large(q, g):
        c, sn = cs(g)

        def body(t, _):
            scl = q - CH_LOG
            blk = t >> scl
            j = t - (blk << scl)
            a0 = (blk << (q + 1)) + (j << CH_LOG)
            b0 = a0 + (1 << q)
            a = _bf(out_ref[pl.ds(a0, CH), :])
            b = _bf(out_ref[pl.ds(b0, CH), :])
            out_ref[pl.ds(a0, CH), :] = c * a - sn * b
            out_ref[pl.ds(b0, CH), :] = sn * a + c * b
            return 0

        fori(0, DIM // (2 * CH), body, 0)

    def cnot_small(ctrl, tgt):
        s = 1 << tgt
        rr = jax.lax.broadcasted_iota(jnp.int32, (CH, B), 0)
        mask = ((rr >> ctrl) & 1) == 1

        def body(t, _):
            r0 = t * CH
            x = out_ref[pl.ds(r0, CH), :]
            v = x.reshape(CH // (2 * s), 2, s, B)
            y = jnp.stack([v[:, 1], v[:, 0]], axis=1).reshape(CH, B)
            out_ref[pl.ds(r0, CH), :] = jnp.where(mask, y, x)
            return 0

        fori(0, NCH, body, 0)

    def cnot_large(i):
        def body(t, _):
            scl = i - CH_LOG
            blk = t >> scl
            j = t - (blk << scl)
            a0 = (blk << (i + 2)) + (1 << i) + (j << CH_LOG)
            b0 = a0 + (2 << i)
            a = out_ref[pl.ds(a0, CH), :]
            b = out_ref[pl.ds(b0, CH), :]
            out_ref[pl.ds(a0, CH), :] = b
            out_ref[pl.ds(b0, CH), :] = a
            return 0

        fori(0, DIM // (4 * CH), body, 0)

    def cnot_14_0():
        def body(t, _):
            r0 = ((t >> 6) << 15) + (1 << 14) + ((t & 63) << CH_LOG)
            x = out_ref[pl.ds(r0, CH), :]
            v = x.reshape(CH // 2, 2, 1, B)
            out_ref[pl.ds(r0, CH), :] = jnp.stack(
                [v[:, 1], v[:, 0]], axis=1).reshape(CH, B)
            return 0

        fori(0, DIM // (2 * CH), body, 0)

    def zero_body(t, _):
        out_ref[pl.ds(t * CH, CH), :] = jnp.zeros((CH, B), jnp.float32)
        return 0

    fori(0, NCH, zero_body, 0)
    r8 = jax.lax.broadcasted_iota(jnp.int32, (8, B), 0)
    out_ref[pl.ds(0, 8), :] = jnp.where(r8 == 0, 1.0, 0.0)

    def ry_pass(base):
        for q in range(CH_LOG):
            ry_small(q, base + q)
        fori(CH_LOG, N_QUBITS, lambda q, _: (ry_large(q, base + q), 0)[1], 0)

    def layer_body(layer, _):
        base = layer * 2 * N_QUBITS
        ry_pass(base)
        for i in range(CH_LOG - 1):
            cnot_small(i, i + 1)
        cnot_ctrl_in_chunk(CH_LOG - 1)
        fori(CH_LOG, N_QUBITS - 1, lambda i, _: (cnot_large(i), 0)[1], 0)
        cnot_14_0()
        ry_pass(base + N_QUBITS)
        return 0

    def cnot_ctrl_in_chunk(i):
        rr = jax.lax.broadcasted_iota(jnp.int32, (CH, B), 0)
        mask = ((rr >> i) & 1) == 1

        def body(t, _):
            scl = i + 1 - CH_LOG
            blk = t >> scl
            j = t - (blk << scl)
            a0 = (blk << (i + 2)) + (j << CH_LOG)
            b0 = a0 + (2 << i)
            a = out_ref[pl.ds(a0, CH), :]
            b = out_ref[pl.ds(b0, CH), :]
            out_ref[pl.ds(a0, CH), :] = jnp.where(mask, b, a)
            out_ref[pl.ds(b0, CH), :] = jnp.where(mask, a, b)
            return 0

        fori(0, DIM // (2 * CH), body, 0)

    fori(0, N_LAYERS, layer_body, 0)

    def sq_body(t, _):
        x = out_ref[pl.ds(t * CH, CH), :]
        out_ref[pl.ds(t * CH, CH), :] = x * x
        return 0

    fori(0, NCH, sq_body, 0)


KB = 4096
NKB = DIM // KB


def _mlp_body(p_ref, w1_ref, w2_ref, w3_ref, b1_ref, b2_ref, b3_ref,
              out_ref, acc_ref):
    k = pl.program_id(0)

    @pl.when(k == 0)
    def _():
        acc_ref[:, :] = jnp.zeros_like(acc_ref)

    acc_ref[:, :] += jax.lax.dot_general(
        w1_ref[:, :], p_ref[:, :], (((1,), (0,)), ((), ())),
        preferred_element_type=jnp.float32,
        precision=jax.lax.Precision.HIGHEST)

    @pl.when(k == NKB - 1)
    def _():
        h1 = jnp.maximum(acc_ref[:, :] + b1_ref[:, :], 0.0)
        h2 = jnp.maximum(
            jax.lax.dot_general(w2_ref[:, :], h1, (((1,), (0,)), ((), ())),
                                preferred_element_type=jnp.float32,
                                precision=jax.lax.Precision.HIGHEST)
            + b2_ref[:, :], 0.0)
        out_ref[:, :] = jax.lax.dot_general(
            w3_ref[:, :], h2, (((1,), (0,)), ((), ())),
            preferred_element_type=jnp.float32,
            precision=jax.lax.Precision.HIGHEST) + b3_ref[:, :]


@jax.jit
def kernel(input_features, theta_params, phi_params, W1, b1, W2, b2, W3, b3):
    ft = input_features.T
    layers = []
    for layer in range(N_LAYERS):
        layers.append(theta_params[layer][:, None] + 0.1 * ft[:N_QUBITS])
        layers.append(phi_params[layer][:, None] + 0.1 * ft[2:2 + N_QUBITS])
    ang = 0.5 * jnp.concatenate(layers, axis=0)[:, None, :]

    probs = pl.pallas_call(
        _quantum_body,
        out_shape=jax.ShapeDtypeStruct((DIM, B), jnp.float32),
    )(jnp.cos(ang), jnp.sin(ang))

    out = pl.pallas_call(
        _mlp_body,
        grid=(NKB,),
        in_specs=[
            pl.BlockSpec((KB, B), lambda k: (k, 0)),
            pl.BlockSpec((512, KB), lambda k: (0, k)),
            pl.BlockSpec((256, 512), lambda k: (0, 0)),
            pl.BlockSpec((128, 256), lambda k: (0, 0)),
            pl.BlockSpec((512, 1), lambda k: (0, 0)),
            pl.BlockSpec((256, 1), lambda k: (0, 0)),
            pl.BlockSpec((128, 1), lambda k: (0, 0)),
        ],
        out_specs=pl.BlockSpec((128, B), lambda k: (0, 0)),
        out_shape=jax.ShapeDtypeStruct((128, B), jnp.float32),
        scratch_shapes=[pltpu.VMEM((512, B), jnp.float32)],
        compiler_params=pltpu.CompilerParams(
            dimension_semantics=("arbitrary",)),
    )(probs, W1, W2, W3, b1[:, None], b2[:, None], b3[:, None])

    return out.T


# speedup vs baseline: 9.0092x; 1.0279x over previous
import jax
import jax.numpy as jnp
from jax.experimental import pallas as pl
from jax.experimental.pallas import tpu as pltpu
from functools import partial

N_QUBITS = 15
N_LAYERS = 3
DIM = 2 ** N_QUBITS
B = 128
PAIRS = [(i, (i + 1) % N_QUBITS) for i in range(N_QUBITS)]


def _bf(x):
    return x.astype(jnp.bfloat16).astype(jnp.float32)


CH = 512
CH_LOG = 9
NCH = DIM // CH


def _quantum_body(cos_ref, sin_ref, out_ref):
    fori = jax.lax.fori_loop

    def cs(g):
        return _bf(cos_ref[g]), _bf(sin_ref[g])

    def ry_small(q, g):
        c, sn = cs(g)
        s = 1 << q

        def body(t, _):
            r0 = t * CH
            x = out_ref[pl.ds(r0, CH), :]
            v = x.reshape(CH // (2 * s), 2, s, B)
            a = _bf(v[:, 0])
            b = _bf(v[:, 1])
            na = c * a - sn * b
            nb = sn * a + c * b
            out_ref[pl.ds(r0, CH), :] = jnp.stack(
                [na, nb], axis=1).reshape(CH, B)
            return 0

        fori(0, NCH, body, 0)

    def ry_large(q, g):
        c, sn = cs(g)

        def body(t, _):
            scl = q - CH_LOG
            blk = t >> scl
            j = t - (blk << scl)
            a0 = (blk << (q + 1)) + (j << CH_LOG)
            b0 = a0 + (1 << q)
            a = _bf(out_ref[pl.ds(a0, CH), :])
            b = _bf(out_ref[pl.ds(b0, CH), :])
            out_ref[pl.ds(a0, CH), :] = c * a - sn * b
            out_ref[pl.ds(b0, CH), :] = sn * a + c * b
            return 0

        fori(0, DIM // (2 * CH), body, 0)

    def cnot_small(ctrl, tgt):
        s = 1 << tgt
        rr = jax.lax.broadcasted_iota(jnp.int32, (CH, B), 0)
        mask = ((rr >> ctrl) & 1) == 1

        def body(t, _):
            r0 = t * CH
            x = out_ref[pl.ds(r0, CH), :]
            v = x.reshape(CH // (2 * s), 2, s, B)
            y = jnp.stack([v[:, 1], v[:, 0]], axis=1).reshape(CH, B)
            out_ref[pl.ds(r0, CH), :] = jnp.where(mask, y, x)
            return 0

        fori(0, NCH, body, 0)

    def cnot_large(i):
        def body(t, _):
            scl = i - CH_LOG
            blk = t >> scl
            j = t - (blk << scl)
            a0 = (blk << (i + 2)) + (1 << i) + (j << CH_LOG)
            b0 = a0 + (2 << i)
            a = out_ref[pl.ds(a0, CH), :]
            b = out_ref[pl.ds(b0, CH), :]
            out_ref[pl.ds(a0, CH), :] = b
            out_ref[pl.ds(b0, CH), :] = a
            return 0

        fori(0, DIM // (4 * CH), body, 0)

    def cnot_14_0():
        def body(t, _):
            lb = 14 - CH_LOG
            r0 = ((t >> lb) << 15) + (1 << 14) + ((t & ((1 << lb) - 1)) << CH_LOG)
            x = out_ref[pl.ds(r0, CH), :]
            v = x.reshape(CH // 2, 2, 1, B)
            out_ref[pl.ds(r0, CH), :] = jnp.stack(
                [v[:, 1], v[:, 0]], axis=1).reshape(CH, B)
            return 0

        fori(0, DIM // (2 * CH), body, 0)

    def zero_body(t, _):
        out_ref[pl.ds(t * CH, CH), :] = jnp.zeros((CH, B), jnp.float32)
        return 0

    fori(0, NCH, zero_body, 0)
    r8 = jax.lax.broadcasted_iota(jnp.int32, (8, B), 0)
    out_ref[pl.ds(0, 8), :] = jnp.where(r8 == 0, 1.0, 0.0)

    def ry_pass(base):
        for q in range(CH_LOG):
            ry_small(q, base + q)
        fori(CH_LOG, N_QUBITS, lambda q, _: (ry_large(q, base + q), 0)[1], 0)

    def layer_body(layer, _):
        base = layer * 2 * N_QUBITS
        ry_pass(base)
        for i in range(CH_LOG - 1):
            cnot_small(i, i + 1)
        cnot_ctrl_in_chunk(CH_LOG - 1)
        fori(CH_LOG, N_QUBITS - 1, lambda i, _: (cnot_large(i), 0)[1], 0)
        cnot_14_0()
        ry_pass(base + N_QUBITS)
        return 0

    def cnot_ctrl_in_chunk(i):
        rr = jax.lax.broadcasted_iota(jnp.int32, (CH, B), 0)
        mask = ((rr >> i) & 1) == 1

        def body(t, _):
            scl = i + 1 - CH_LOG
            blk = t >> scl
            j = t - (blk << scl)
            a0 = (blk << (i + 2)) + (j << CH_LOG)
            b0 = a0 + (2 << i)
            a = out_ref[pl.ds(a0, CH), :]
            b = out_ref[pl.ds(b0, CH), :]
            out_ref[pl.ds(a0, CH), :] = jnp.where(mask, b, a)
            out_ref[pl.ds(b0, CH), :] = jnp.where(mask, a, b)
            return 0

        fori(0, DIM // (2 * CH), body, 0)

    fori(0, N_LAYERS, layer_body, 0)

    def sq_body(t, _):
        x = out_ref[pl.ds(t * CH, CH), :]
        out_ref[pl.ds(t * CH, CH), :] = x * x
        return 0

    fori(0, NCH, sq_body, 0)


KB = 4096
NKB = DIM // KB


def _mlp_body(p_ref, w1_ref, w2_ref, w3_ref, b1_ref, b2_ref, b3_ref,
              out_ref, acc_ref):
    k = pl.program_id(0)

    @pl.when(k == 0)
    def _():
        acc_ref[:, :] = jnp.zeros_like(acc_ref)

    acc_ref[:, :] += jax.lax.dot_general(
        w1_ref[:, :], p_ref[:, :], (((1,), (0,)), ((), ())),
        preferred_element_type=jnp.float32,
        precision=jax.lax.Precision.HIGHEST)

    @pl.when(k == NKB - 1)
    def _():
        h1 = jnp.maximum(acc_ref[:, :] + b1_ref[:, :], 0.0)
        h2 = jnp.maximum(
            jax.lax.dot_general(w2_ref[:, :], h1, (((1,), (0,)), ((), ())),
                                preferred_element_type=jnp.float32,
                                precision=jax.lax.Precision.HIGHEST)
            + b2_ref[:, :], 0.0)
        out_ref[:, :] = jax.lax.dot_general(
            w3_ref[:, :], h2, (((1,), (0,)), ((), ())),
            preferred_element_type=jnp.float32,
            precision=jax.lax.Precision.HIGHEST) + b3_ref[:, :]


@jax.jit
def kernel(input_features, theta_params, phi_params, W1, b1, W2, b2, W3, b3):
    ft = input_features.T
    layers = []
    for layer in range(N_LAYERS):
        layers.append(theta_params[layer][:, None] + 0.1 * ft[:N_QUBITS])
        layers.append(phi_params[layer][:, None] + 0.1 * ft[2:2 + N_QUBITS])
    ang = 0.5 * jnp.concatenate(layers, axis=0)[:, None, :]

    probs = pl.pallas_call(
        _quantum_body,
        out_shape=jax.ShapeDtypeStruct((DIM, B), jnp.float32),
    )(jnp.cos(ang), jnp.sin(ang))

    out = pl.pallas_call(
        _mlp_body,
        grid=(NKB,),
        in_specs=[
            pl.BlockSpec((KB, B), lambda k: (k, 0)),
            pl.BlockSpec((512, KB), lambda k: (0, k)),
            pl.BlockSpec((256, 512), lambda k: (0, 0)),
            pl.BlockSpec((128, 256), lambda k: (0, 0)),
            pl.BlockSpec((512, 1), lambda k: (0, 0)),
            pl.BlockSpec((256, 1), lambda k: (0, 0)),
            pl.BlockSpec((128, 1), lambda k: (0, 0)),
        ],
        out_specs=pl.BlockSpec((128, B), lambda k: (0, 0)),
        out_shape=jax.ShapeDtypeStruct((128, B), jnp.float32),
        scratch_shapes=[pltpu.VMEM((512, B), jnp.float32)],
        compiler_params=pltpu.CompilerParams(
            dimension_semantics=("arbitrary",)),
    )(probs, W1, W2, W3, b1[:, None], b2[:, None], b3[:, None])

    return out.T
